# baseline (speedup 1.0000x reference)
import functools

import jax
import jax.numpy as jnp
from jax import lax
from jax.experimental import pallas as pl
from jax.experimental.pallas import tpu as pltpu
from jax.experimental.pallas import tpu_sc as plsc

N0, M0 = 10000, 2500
N1, M1 = 2500, 1000
NNZ0, NNZ1 = 320000, 40000
D = 128

NC, NS, LANES = 2, 16, 16
NW = NC * NS
CHUNK = 128

SP_N0, SP_M0, SP_N1, SP_M1 = 10240, 2560, 2560, 1024

K0 = 80
K1 = 16
KP = 8
P0 = NW * K0 * CHUNK
P1 = NW * K1 * CHUNK
PP = NW * KP * CHUNK


def _mesh():
    return plsc.VectorSubcoreMesh(core_axis_name="c", subcore_axis_name="s",
                                  num_cores=NC, num_subcores=NS)


@functools.lru_cache(None)
def _make_rowpass(k, sp):
    rpt = sp // NS
    ksub = min(k, 40)
    n_half = k // ksub

    @functools.partial(
        pl.kernel,
        out_type=jax.ShapeDtypeStruct((NC, sp, D), jnp.float32),
        mesh=_mesh(),
        scratch_types=[
            pltpu.VMEM((ksub, CHUNK), jnp.int32),
            pltpu.VMEM((ksub, CHUNK), jnp.int32),
            pltpu.VMEM((CHUNK, D), jnp.float32),
            pltpu.VMEM((CHUNK, D), jnp.float32),
            pltpu.VMEM_SHARED((sp, D), jnp.float32),
            pltpu.SemaphoreType.DMA,
            pltpu.SemaphoreType.DMA,
        ],
    )
    def rowpass(table, src2d, dst2d, zeros, out,
                idx_s, idx_d, rows0, rows1, acc, sem0, sem1):
        cid = lax.axis_index("c")
        sid = lax.axis_index("s")
        wid = sid * NC + cid
        pltpu.sync_copy(zeros.at[pl.ds(sid * rpt, rpt)],
                        acc.at[pl.ds(sid * rpt, rpt)])
        plsc.subcore_barrier()

        rows = (rows0, rows1)
        sems = (sem0, sem1)
        for h in range(n_half):
            pltpu.sync_copy(src2d.at[pl.ds(wid * k + h * ksub, ksub)], idx_s)
            pltpu.sync_copy(dst2d.at[pl.ds(wid * k + h * ksub, ksub)], idx_d)
            pltpu.async_copy(table.at[idx_s.at[0]], rows0, sem0)

            @pl.loop(0, ksub // 2)
            def _(i):
                for u in (0, 1):
                    t = i * 2 + u
                    pltpu.make_async_copy(table.at[idx_s.at[t]],
                                          rows[u], sems[u]).wait()

                    @pl.when(t + 1 < ksub)
                    def _():
                        pltpu.async_copy(table.at[idx_s.at[t + 1]],
                                         rows[1 - u], sems[1 - u])

                    pltpu.sync_copy(rows[u], acc.at[idx_d.at[t]], add=True)

        plsc.subcore_barrier()
        pltpu.sync_copy(acc.at[pl.ds(sid * rpt, rpt)],
                        out.at[cid, pl.ds(sid * rpt, rpt)])

    return rowpass


@functools.lru_cache(None)
def _make_rowpass4(k, sp):
    rpt = sp // NS

    @functools.partial(
        pl.kernel,
        out_type=jax.ShapeDtypeStruct((NC, sp, D), jnp.float32),
        mesh=_mesh(),
        scratch_types=[
            pltpu.VMEM((k, CHUNK), jnp.int32),
            pltpu.VMEM((k, CHUNK), jnp.int32),
            pltpu.VMEM((CHUNK, D), jnp.float32),
            pltpu.VMEM((CHUNK, D), jnp.float32),
            pltpu.VMEM((CHUNK, D), jnp.float32),
            pltpu.VMEM((CHUNK, D), jnp.float32),
            pltpu.VMEM_SHARED((sp, D), jnp.float32),
            pltpu.SemaphoreType.DMA,
            pltpu.SemaphoreType.DMA,
            pltpu.SemaphoreType.DMA,
            pltpu.SemaphoreType.DMA,
            pltpu.SemaphoreType.DMA,
            pltpu.SemaphoreType.DMA,
            pltpu.SemaphoreType.DMA,
            pltpu.SemaphoreType.DMA,
        ],
    )
    def rowpass(table, src2d, dst2d, zeros, out,
                idx_s, idx_d, r0, r1, r2, r3, acc,
                g0, g1, g2, g3, s0, s1, s2, s3):
        cid = lax.axis_index("c")
        sid = lax.axis_index("s")
        wid = sid * NC + cid
        rows = (r0, r1, r2, r3)
        gsem = (g0, g1, g2, g3)
        ssem = (s0, s1, s2, s3)
        pltpu.sync_copy(zeros.at[pl.ds(sid * rpt, rpt)],
                        acc.at[pl.ds(sid * rpt, rpt)])
        pltpu.sync_copy(src2d.at[pl.ds(wid * k, k)], idx_s)
        pltpu.sync_copy(dst2d.at[pl.ds(wid * k, k)], idx_d)
        plsc.subcore_barrier()

        pltpu.async_copy(table.at[idx_s.at[0]], rows[0], gsem[0])
        pltpu.async_copy(table.at[idx_s.at[1]], rows[1], gsem[1])

        @pl.loop(0, k // 4)
        def _(i):
            for u in range(4):
                t = i * 4 + u
                bf = (u + 2) % 4
                pltpu.make_async_copy(table.at[idx_s.at[t]],
                                      rows[u], gsem[u]).wait()
                pltpu.async_copy(rows[u], acc.at[idx_d.at[t]],
                                 ssem[u], add=True)

                @pl.when(t + 2 < k)
                def _():
                    @pl.when(t >= 2)
                    def _():
                        pltpu.make_async_copy(
                            rows[bf], acc.at[idx_d.at[t]], ssem[bf]).wait()

                    pltpu.async_copy(table.at[idx_s.at[t + 2]],
                                     rows[bf], gsem[bf])

        for b in range(4):
            pltpu.make_async_copy(rows[b], acc.at[idx_d.at[0]],
                                  ssem[b]).wait()
        plsc.subcore_barrier()
        pltpu.sync_copy(acc.at[pl.ds(sid * rpt, rpt)],
                        out.at[cid, pl.ds(sid * rpt, rpt)])

    return rowpass


_HISTS = ((SP_N0, P0), (SP_M0, P0), (SP_N1, P1), (SP_M1, P1), (SP_M0, PP))


@functools.lru_cache(None)
def _make_hist():
    pmax = max(p for _, p in _HISTS) // NS

    @functools.partial(
        pl.kernel,
        out_type=[jax.ShapeDtypeStruct((NC * sp,), jnp.float32)
                  for sp, _ in _HISTS],
        mesh=_mesh(),
        compiler_params=pltpu.CompilerParams(needs_layout_passes=False),
        scratch_types=[
            pltpu.VMEM((pmax,), jnp.int32),
            pltpu.VMEM((max(sp for sp, _ in _HISTS) // NS,), jnp.float32),
            pltpu.VMEM((max(sp for sp, _ in _HISTS) // NS,), jnp.float32),
        ] + [pltpu.VMEM((sp,), jnp.float32) for sp, _ in _HISTS]
          + [pltpu.VMEM_SHARED((NS * sp,), jnp.float32) for sp, _ in _HISTS],
    )
    def hist(i0, i1, i2, i3, i4,
             o0, o1, o2, o3, o4,
             idxbuf, tmp, tmp2, a0, a1, a2, a3, a4,
             s0, s1, s2, s3, s4):
        cid = lax.axis_index("c")
        sid = lax.axis_index("s")
        ones_v = jnp.ones((LANES,), jnp.float32)
        zeros_v = jnp.zeros((LANES,), jnp.float32)
        accs = (a0, a1, a2, a3, a4)
        stages = (s0, s1, s2, s3, s4)
        outs = (o0, o1, o2, o3, o4)
        idxs = (i0, i1, i2, i3, i4)

        for idx_flat, a, st, (sp, p) in zip(idxs, accs, stages, _HISTS):
            ppt = p // (NS * NC)

            @pl.loop(0, sp // LANES)
            def _(v, a=a):
                a[pl.ds(v * LANES, LANES)] = zeros_v

            pltpu.sync_copy(
                idx_flat.at[pl.ds(cid * (p // NC) + sid * ppt, ppt)],
                idxbuf.at[pl.ds(0, ppt)])

            @pl.loop(0, ppt // LANES)
            def _(q, a=a):
                vec = idxbuf[pl.ds(q * LANES, LANES)]
                plsc.addupdate_scatter(a, [vec], ones_v)

            pltpu.sync_copy(a, st.at[pl.ds(sid * sp, sp)])

        plsc.subcore_barrier()

        for st, o, (sp, _) in zip(stages, outs, _HISTS):
            spt = sp // NS
            pltpu.sync_copy(st.at[pl.ds(sid * spt, spt)],
                            tmp.at[pl.ds(0, spt)])
            for s in range(1, NS):
                pltpu.sync_copy(st.at[pl.ds(s * sp + sid * spt, spt)],
                                tmp2.at[pl.ds(0, spt)])

                @pl.loop(0, spt // LANES)
                def _(v):
                    sl = pl.ds(v * LANES, LANES)
                    tmp[sl] = tmp[sl] + tmp2[sl]

            pltpu.sync_copy(tmp.at[pl.ds(0, spt)],
                            o.at[pl.ds(cid * sp + sid * spt, spt)])

    return hist


@functools.lru_cache(None)
def _make_unpool(k):
    @functools.partial(
        pl.kernel,
        out_type=jax.ShapeDtypeStruct((NW * k * CHUNK, D), jnp.float32),
        mesh=_mesh(),
        scratch_types=[
            pltpu.VMEM((k, CHUNK), jnp.int32),
            pltpu.VMEM((CHUNK, D), jnp.float32),
            pltpu.SemaphoreType.DMA,
        ],
    )
    def unpool(table, src2d, out, idx_s, rows, sem):
        cid = lax.axis_index("c")
        sid = lax.axis_index("s")
        wid = sid * NC + cid
        pltpu.sync_copy(src2d.at[pl.ds(wid * k, k)], idx_s)

        @pl.loop(0, k)
        def _(j):
            pltpu.async_copy(table.at[idx_s.at[j]], rows, sem).wait()
            pltpu.sync_copy(rows, out.at[pl.ds((wid * k + j) * CHUNK, CHUNK)])

    return unpool


def _tc(body, out_shape, *args):
    return pl.pallas_call(body, out_shape=out_shape)(*args)


def _safe_rsqrt(c):
    return jnp.where(c > 0, lax.rsqrt(jnp.where(c > 0, c, 1.0)), 0.0)


def _safe_recip(c):
    return jnp.where(c > 0, 1.0 / jnp.where(c > 0, c, 1.0), 0.0)


def _comb_body(hn0, hh0, hn1, hh1, hasm, dvs0, dei0, dvs1, dei1, ci):
    dvs0[...] = _safe_rsqrt(hn0[0] + hn0[1])
    dei0[...] = _safe_recip(hh0[0] + hh0[1])
    dvs1[...] = _safe_rsqrt(hn1[0] + hn1[1])
    dei1[...] = _safe_recip(hh1[0] + hh1[1])
    ci[...] = _safe_recip(hasm[0] + hasm[1])


def _mm_scale_body(x, w, b, s, o):
    o[...] = (jnp.dot(x[...], w[...], preferred_element_type=jnp.float32)
              + b[...]) * s[...]


def _comb_scale_body(p, s, o):
    o[...] = (p[0] + p[1]) * s[...]


def _comb_scale_relu_body(p, s, o):
    o[...] = jnp.maximum((p[0] + p[1]) * s[...], 0.0)


def _pool_mm_body(p, ci, w, b, s, o):
    xc = (p[0] + p[1]) * ci[...]
    o[...] = (jnp.dot(xc, w[...], preferred_element_type=jnp.float32)
              + b[...]) * s[...]


def _skip_mm_body(xu, h0, wa, wb, b, s, o):
    y = (jnp.dot(xu[0:SP_N0, :], wa[...], preferred_element_type=jnp.float32)
         + jnp.dot(h0[...], wb[...], preferred_element_type=jnp.float32)
         + b[...])
    o[...] = y * s[...]


def _final_body(p, s, w, b, o):
    h2 = jnp.maximum((p[0] + p[1]) * s[...], 0.0)
    o[...] = jnp.dot(h2, w[...], preferred_element_type=jnp.float32) + b[...]


def _f32(shape):
    return jax.ShapeDtypeStruct(shape, jnp.float32)


def _pad_idx(a, total, fill):
    pad = jnp.full((total - a.shape[0],), fill, jnp.int32)
    return jnp.concatenate([a.astype(jnp.int32), pad]).reshape(-1, CHUNK)


def kernel(X, W0, b0, W1, b1, W2, b2, W3, b3,
           inc_nodes0, inc_hedges0, inc_nodes1, inc_hedges1, asm_idx):
    f32 = jnp.float32
    zeros128 = jnp.zeros((SP_N0, D), f32)
    zeros16 = jnp.zeros((SP_N0, LANES), f32)

    srcA0 = _pad_idx(inc_nodes0, P0, 0)
    dstA0 = _pad_idx(inc_hedges0, P0, M0)
    srcB0 = _pad_idx(inc_hedges0, P0, 0)
    dstB0 = _pad_idx(inc_nodes0, P0, N0)
    srcA1 = _pad_idx(inc_nodes1, P1, 0)
    dstA1 = _pad_idx(inc_hedges1, P1, M1)
    srcB1 = _pad_idx(inc_hedges1, P1, 0)
    dstB1 = _pad_idx(inc_nodes1, P1, N1)
    srcP = _pad_idx(jnp.arange(N0, dtype=jnp.int32), PP, 0)
    dstP = _pad_idx(asm_idx, PP, N1)
    srcU = _pad_idx(asm_idx, PP, 0)

    Xp = jnp.concatenate([X, jnp.zeros((SP_N0 - N0, D), f32)], axis=0)
    b0r, b1r, b2r, b3r = (b.reshape(1, D) for b in (b0, b1, b2, b3))

    hn0, hh0, hn1, hh1, hasm = _make_hist()(
        dstB0.reshape(-1), dstA0.reshape(-1), dstB1.reshape(-1),
        dstA1.reshape(-1), dstP.reshape(-1))
    dvs0, dei0, dvs1, dei1, ci = _tc(
        _comb_body,
        (_f32((SP_N0, 1)), _f32((SP_M0, 1)), _f32((SP_N1, 1)),
         _f32((SP_M1, 1)), _f32((SP_M0, 1))),
        hn0.reshape(NC, -1, 1), hh0.reshape(NC, -1, 1),
        hn1.reshape(NC, -1, 1), hh1.reshape(NC, -1, 1),
        hasm.reshape(NC, -1, 1))

    zrows = lambda sp: zeros128[:sp]

    Xs0 = _tc(_mm_scale_body, _f32((SP_N0, D)), Xp, W0, b0r, dvs0)
    pA = _make_rowpass4(K0, SP_M0)(Xs0, srcA0, dstA0, zrows(SP_M0))
    he0 = _tc(_comb_scale_body, _f32((SP_M0, D)), pA, dei0)
    pB = _make_rowpass(K0, SP_N0)(he0, srcB0, dstB0, zrows(SP_N0))
    h0 = _tc(_comb_scale_relu_body, _f32((SP_N0, D)), pB, dvs0)

    pP = _make_rowpass4(KP, SP_M0)(h0, srcP, dstP, zrows(SP_M0))
    Xs1 = _tc(_pool_mm_body, _f32((SP_N1, D)), pP, ci, W1, b1r, dvs1)
    qA = _make_rowpass4(K1, SP_M1)(Xs1, srcA1, dstA1, zrows(SP_M1))
    he1 = _tc(_comb_scale_body, _f32((SP_M1, D)), qA, dei1)
    qB = _make_rowpass4(K1, SP_N1)(he1, srcB1, dstB1, zrows(SP_N1))
    hb = _tc(_comb_scale_relu_body, _f32((SP_N1, D)), qB, dvs1)

    Xu = _make_unpool(KP)(hb, srcU)
    W2a, W2b = W2[:D], W2[D:]
    Y2 = _tc(_skip_mm_body, _f32((SP_N0, D)), Xu, h0, W2a, W2b, b2r, dvs0)
    rA = _make_rowpass4(K0, SP_M0)(Y2, srcA0, dstA0, zrows(SP_M0))
    he2 = _tc(_comb_scale_body, _f32((SP_M0, D)), rA, dei0)
    rB = _make_rowpass(K0, SP_N0)(he2, srcB0, dstB0, zrows(SP_N0))
    out = _tc(_final_body, _f32((SP_N0, D)), rB, dvs0, W3, b3r)

    return out[:N0]


# speedup vs baseline: 1.0232x; 1.0232x over previous
import functools

import jax
import jax.numpy as jnp
from jax import lax
from jax.experimental import pallas as pl
from jax.experimental.pallas import tpu as pltpu
from jax.experimental.pallas import tpu_sc as plsc

N0, M0 = 10000, 2500
N1, M1 = 2500, 1000
NNZ0, NNZ1 = 320000, 40000
D = 128

NC, NS, LANES = 2, 16, 16
NW = NC * NS
CHUNK = 128

SP_N0, SP_M0, SP_N1, SP_M1 = 10240, 2560, 2560, 1024

K0 = 80
K1 = 16
KP = 8
P0 = NW * K0 * CHUNK
P1 = NW * K1 * CHUNK
PP = NW * KP * CHUNK


def _mesh():
    return plsc.VectorSubcoreMesh(core_axis_name="c", subcore_axis_name="s",
                                  num_cores=NC, num_subcores=NS)


@functools.lru_cache(None)
def _make_rowpass(k, sp):
    rpt = sp // NS
    ksub = min(k, 40)
    n_half = k // ksub

    @functools.partial(
        pl.kernel,
        out_type=jax.ShapeDtypeStruct((NC, sp, D), jnp.float32),
        mesh=_mesh(),
        scratch_types=[
            pltpu.VMEM((ksub, CHUNK), jnp.int32),
            pltpu.VMEM((ksub, CHUNK), jnp.int32),
            pltpu.VMEM((CHUNK, D), jnp.float32),
            pltpu.VMEM((CHUNK, D), jnp.float32),
            pltpu.VMEM_SHARED((sp, D), jnp.float32),
            pltpu.SemaphoreType.DMA,
            pltpu.SemaphoreType.DMA,
        ],
    )
    def rowpass(table, src2d, dst2d, zeros, out,
                idx_s, idx_d, rows0, rows1, acc, sem0, sem1):
        cid = lax.axis_index("c")
        sid = lax.axis_index("s")
        wid = sid * NC + cid
        pltpu.sync_copy(zeros.at[pl.ds(sid * rpt, rpt)],
                        acc.at[pl.ds(sid * rpt, rpt)])
        plsc.subcore_barrier()

        rows = (rows0, rows1)
        sems = (sem0, sem1)
        for h in range(n_half):
            pltpu.sync_copy(src2d.at[pl.ds(wid * k + h * ksub, ksub)], idx_s)
            pltpu.sync_copy(dst2d.at[pl.ds(wid * k + h * ksub, ksub)], idx_d)
            pltpu.async_copy(table.at[idx_s.at[0]], rows0, sem0)

            @pl.loop(0, ksub // 2)
            def _(i):
                for u in (0, 1):
                    t = i * 2 + u
                    pltpu.make_async_copy(table.at[idx_s.at[t]],
                                          rows[u], sems[u]).wait()

                    @pl.when(t + 1 < ksub)
                    def _():
                        pltpu.async_copy(table.at[idx_s.at[t + 1]],
                                         rows[1 - u], sems[1 - u])

                    pltpu.sync_copy(rows[u], acc.at[idx_d.at[t]], add=True)

        plsc.subcore_barrier()
        pltpu.sync_copy(acc.at[pl.ds(sid * rpt, rpt)],
                        out.at[cid, pl.ds(sid * rpt, rpt)])

    return rowpass


@functools.lru_cache(None)
def _make_rowpass4(k, sp):
    rpt = sp // NS

    @functools.partial(
        pl.kernel,
        out_type=jax.ShapeDtypeStruct((NC, sp, D), jnp.float32),
        mesh=_mesh(),
        scratch_types=[
            pltpu.VMEM((k, CHUNK), jnp.int32),
            pltpu.VMEM((k, CHUNK), jnp.int32),
            pltpu.VMEM((CHUNK, D), jnp.float32),
            pltpu.VMEM((CHUNK, D), jnp.float32),
            pltpu.VMEM((CHUNK, D), jnp.float32),
            pltpu.VMEM((CHUNK, D), jnp.float32),
            pltpu.VMEM_SHARED((sp, D), jnp.float32),
            pltpu.SemaphoreType.DMA,
            pltpu.SemaphoreType.DMA,
            pltpu.SemaphoreType.DMA,
            pltpu.SemaphoreType.DMA,
            pltpu.SemaphoreType.DMA,
            pltpu.SemaphoreType.DMA,
            pltpu.SemaphoreType.DMA,
            pltpu.SemaphoreType.DMA,
        ],
    )
    def rowpass(table, src2d, dst2d, zeros, out,
                idx_s, idx_d, r0, r1, r2, r3, acc,
                g0, g1, g2, g3, s0, s1, s2, s3):
        cid = lax.axis_index("c")
        sid = lax.axis_index("s")
        wid = sid * NC + cid
        rows = (r0, r1, r2, r3)
        gsem = (g0, g1, g2, g3)
        ssem = (s0, s1, s2, s3)
        pltpu.sync_copy(zeros.at[pl.ds(sid * rpt, rpt)],
                        acc.at[pl.ds(sid * rpt, rpt)])
        pltpu.sync_copy(src2d.at[pl.ds(wid * k, k)], idx_s)
        pltpu.sync_copy(dst2d.at[pl.ds(wid * k, k)], idx_d)
        plsc.subcore_barrier()

        pltpu.async_copy(table.at[idx_s.at[0]], rows[0], gsem[0])
        pltpu.async_copy(table.at[idx_s.at[1]], rows[1], gsem[1])

        @pl.loop(0, k // 4)
        def _(i):
            for u in range(4):
                t = i * 4 + u
                bf = (u + 2) % 4
                pltpu.make_async_copy(table.at[idx_s.at[t]],
                                      rows[u], gsem[u]).wait()
                pltpu.async_copy(rows[u], acc.at[idx_d.at[t]],
                                 ssem[u], add=True)

                @pl.when(t + 2 < k)
                def _():
                    @pl.when(t >= 2)
                    def _():
                        pltpu.make_async_copy(
                            rows[bf], acc.at[idx_d.at[t]], ssem[bf]).wait()

                    pltpu.async_copy(table.at[idx_s.at[t + 2]],
                                     rows[bf], gsem[bf])

        for b in range(4):
            pltpu.make_async_copy(rows[b], acc.at[idx_d.at[0]],
                                  ssem[b]).wait()
        plsc.subcore_barrier()
        pltpu.sync_copy(acc.at[pl.ds(sid * rpt, rpt)],
                        out.at[cid, pl.ds(sid * rpt, rpt)])

    return rowpass


_HISTS = ((SP_N0, P0), (SP_M0, P0), (SP_N1, P1), (SP_M1, P1), (SP_M0, PP))


@functools.lru_cache(None)
def _make_hist():
    pmax = max(p for _, p in _HISTS) // NS

    @functools.partial(
        pl.kernel,
        out_type=[jax.ShapeDtypeStruct((sp,), jnp.float32)
                  for sp, _ in _HISTS],
        mesh=_mesh(),
        compiler_params=pltpu.CompilerParams(needs_layout_passes=False),
        scratch_types=[
            pltpu.VMEM((pmax,), jnp.int32),
            pltpu.VMEM((max(sp for sp, _ in _HISTS) // NS,), jnp.float32),
            pltpu.VMEM((max(sp for sp, _ in _HISTS) // NS,), jnp.float32),
        ] + [pltpu.VMEM((sp,), jnp.float32) for sp, _ in _HISTS]
          + [pltpu.VMEM_SHARED((NS * sp,), jnp.float32) for sp, _ in _HISTS],
    )
    def hist(i0, i1, i2, i3, i4,
             o0, o1, o2, o3, o4,
             idxbuf, tmp, tmp2, a0, a1, a2, a3, a4,
             s0, s1, s2, s3, s4):
        cid = lax.axis_index("c")
        sid = lax.axis_index("s")
        ones_v = jnp.ones((LANES,), jnp.float32)
        zeros_v = jnp.zeros((LANES,), jnp.float32)
        accs = (a0, a1, a2, a3, a4)
        stages = (s0, s1, s2, s3, s4)
        outs = (o0, o1, o2, o3, o4)
        idxs = (i0, i1, i2, i3, i4)

        @pl.when(cid == 0)
        def _():
            for idx_flat, a, st, (sp, p) in zip(idxs, accs, stages, _HISTS):
                ppt = p // NS

                @pl.loop(0, sp // LANES)
                def _(v, a=a):
                    a[pl.ds(v * LANES, LANES)] = zeros_v

                pltpu.sync_copy(idx_flat.at[pl.ds(sid * ppt, ppt)],
                                idxbuf.at[pl.ds(0, ppt)])

                @pl.loop(0, ppt // LANES)
                def _(q, a=a):
                    vec = idxbuf[pl.ds(q * LANES, LANES)]
                    plsc.addupdate_scatter(a, [vec], ones_v)

                pltpu.sync_copy(a, st.at[pl.ds(sid * sp, sp)])

        plsc.subcore_barrier()

        @pl.when(cid == 0)
        def _():
            for st, o, (sp, _) in zip(stages, outs, _HISTS):
                spt = sp // NS
                pltpu.sync_copy(st.at[pl.ds(sid * spt, spt)],
                                tmp.at[pl.ds(0, spt)])
                for s in range(1, NS):
                    pltpu.sync_copy(st.at[pl.ds(s * sp + sid * spt, spt)],
                                    tmp2.at[pl.ds(0, spt)])

                    @pl.loop(0, spt // LANES)
                    def _(v):
                        sl = pl.ds(v * LANES, LANES)
                        tmp[sl] = tmp[sl] + tmp2[sl]

                pltpu.sync_copy(tmp.at[pl.ds(0, spt)],
                                o.at[pl.ds(sid * spt, spt)])

    return hist


@functools.lru_cache(None)
def _make_unpool(k):
    @functools.partial(
        pl.kernel,
        out_type=jax.ShapeDtypeStruct((NW * k * CHUNK, D), jnp.float32),
        mesh=_mesh(),
        scratch_types=[
            pltpu.VMEM((k, CHUNK), jnp.int32),
            pltpu.VMEM((CHUNK, D), jnp.float32),
            pltpu.SemaphoreType.DMA,
        ],
    )
    def unpool(table, src2d, out, idx_s, rows, sem):
        cid = lax.axis_index("c")
        sid = lax.axis_index("s")
        wid = sid * NC + cid
        pltpu.sync_copy(src2d.at[pl.ds(wid * k, k)], idx_s)

        @pl.loop(0, k)
        def _(j):
            pltpu.async_copy(table.at[idx_s.at[j]], rows, sem).wait()
            pltpu.sync_copy(rows, out.at[pl.ds((wid * k + j) * CHUNK, CHUNK)])

    return unpool


def _tc(body, out_shape, *args):
    return pl.pallas_call(body, out_shape=out_shape)(*args)


def _safe_rsqrt(c):
    return jnp.where(c > 0, lax.rsqrt(jnp.where(c > 0, c, 1.0)), 0.0)


def _safe_recip(c):
    return jnp.where(c > 0, 1.0 / jnp.where(c > 0, c, 1.0), 0.0)


def _comb_body(hn0, hh0, hn1, hh1, hasm, dvs0, dei0, dvs1, dei1, ci):
    dvs0[...] = _safe_rsqrt(hn0[...])
    dei0[...] = _safe_recip(hh0[...])
    dvs1[...] = _safe_rsqrt(hn1[...])
    dei1[...] = _safe_recip(hh1[...])
    ci[...] = _safe_recip(hasm[...])


def _mm_scale_body(x, w, b, s, o):
    o[...] = (jnp.dot(x[...], w[...], preferred_element_type=jnp.float32)
              + b[...]) * s[...]


def _comb_scale_body(p, s, o):
    o[...] = (p[0] + p[1]) * s[...]


def _comb_scale_relu_body(p, s, o):
    o[...] = jnp.maximum((p[0] + p[1]) * s[...], 0.0)


def _pool_mm_body(p, ci, w, b, s, o):
    xc = (p[0] + p[1]) * ci[...]
    o[...] = (jnp.dot(xc, w[...], preferred_element_type=jnp.float32)
              + b[...]) * s[...]


def _skip_mm_body(xu, h0, wa, wb, b, s, o):
    y = (jnp.dot(xu[0:SP_N0, :], wa[...], preferred_element_type=jnp.float32)
         + jnp.dot(h0[...], wb[...], preferred_element_type=jnp.float32)
         + b[...])
    o[...] = y * s[...]


def _final_body(p, s, w, b, o):
    h2 = jnp.maximum((p[0] + p[1]) * s[...], 0.0)
    o[...] = jnp.dot(h2, w[...], preferred_element_type=jnp.float32) + b[...]


def _f32(shape):
    return jax.ShapeDtypeStruct(shape, jnp.float32)


def _pad_idx(a, total, fill):
    pad = jnp.full((total - a.shape[0],), fill, jnp.int32)
    return jnp.concatenate([a.astype(jnp.int32), pad]).reshape(-1, CHUNK)


def kernel(X, W0, b0, W1, b1, W2, b2, W3, b3,
           inc_nodes0, inc_hedges0, inc_nodes1, inc_hedges1, asm_idx):
    f32 = jnp.float32
    zeros128 = jnp.zeros((SP_N0, D), f32)
    zeros16 = jnp.zeros((SP_N0, LANES), f32)

    srcA0 = _pad_idx(inc_nodes0, P0, 0)
    dstA0 = _pad_idx(inc_hedges0, P0, M0)
    srcB0 = _pad_idx(inc_hedges0, P0, 0)
    dstB0 = _pad_idx(inc_nodes0, P0, N0)
    srcA1 = _pad_idx(inc_nodes1, P1, 0)
    dstA1 = _pad_idx(inc_hedges1, P1, M1)
    srcB1 = _pad_idx(inc_hedges1, P1, 0)
    dstB1 = _pad_idx(inc_nodes1, P1, N1)
    srcP = _pad_idx(jnp.arange(N0, dtype=jnp.int32), PP, 0)
    dstP = _pad_idx(asm_idx, PP, N1)
    srcU = _pad_idx(asm_idx, PP, 0)

    Xp = jnp.concatenate([X, jnp.zeros((SP_N0 - N0, D), f32)], axis=0)
    b0r, b1r, b2r, b3r = (b.reshape(1, D) for b in (b0, b1, b2, b3))

    hn0, hh0, hn1, hh1, hasm = _make_hist()(
        dstB0.reshape(-1), dstA0.reshape(-1), dstB1.reshape(-1),
        dstA1.reshape(-1), dstP.reshape(-1))
    dvs0, dei0, dvs1, dei1, ci = _tc(
        _comb_body,
        (_f32((SP_N0, 1)), _f32((SP_M0, 1)), _f32((SP_N1, 1)),
         _f32((SP_M1, 1)), _f32((SP_M0, 1))),
        hn0.reshape(-1, 1), hh0.reshape(-1, 1), hn1.reshape(-1, 1),
        hh1.reshape(-1, 1), hasm.reshape(-1, 1))

    zrows = lambda sp: zeros128[:sp]

    Xs0 = _tc(_mm_scale_body, _f32((SP_N0, D)), Xp, W0, b0r, dvs0)
    pA = _make_rowpass4(K0, SP_M0)(Xs0, srcA0, dstA0, zrows(SP_M0))
    he0 = _tc(_comb_scale_body, _f32((SP_M0, D)), pA, dei0)
    pB = _make_rowpass(K0, SP_N0)(he0, srcB0, dstB0, zrows(SP_N0))
    h0 = _tc(_comb_scale_relu_body, _f32((SP_N0, D)), pB, dvs0)

    pP = _make_rowpass4(KP, SP_M0)(h0, srcP, dstP, zrows(SP_M0))
    Xs1 = _tc(_pool_mm_body, _f32((SP_N1, D)), pP, ci, W1, b1r, dvs1)
    qA = _make_rowpass4(K1, SP_M1)(Xs1, srcA1, dstA1, zrows(SP_M1))
    he1 = _tc(_comb_scale_body, _f32((SP_M1, D)), qA, dei1)
    qB = _make_rowpass4(K1, SP_N1)(he1, srcB1, dstB1, zrows(SP_N1))
    hb = _tc(_comb_scale_relu_body, _f32((SP_N1, D)), qB, dvs1)

    Xu = _make_unpool(KP)(hb, srcU)
    W2a, W2b = W2[:D], W2[D:]
    Y2 = _tc(_skip_mm_body, _f32((SP_N0, D)), Xu, h0, W2a, W2b, b2r, dvs0)
    rA = _make_rowpass4(K0, SP_M0)(Y2, srcA0, dstA0, zrows(SP_M0))
    he2 = _tc(_comb_scale_body, _f32((SP_M0, D)), rA, dei0)
    rB = _make_rowpass(K0, SP_N0)(he2, srcB0, dstB0, zrows(SP_N0))
    out = _tc(_final_body, _f32((SP_N0, D)), rB, dvs0, W3, b3r)

    return out[:N0]


# speedup vs baseline: 1.0236x; 1.0003x over previous
import functools

import jax
import jax.numpy as jnp
from jax import lax
from jax.experimental import pallas as pl
from jax.experimental.pallas import tpu as pltpu
from jax.experimental.pallas import tpu_sc as plsc

N0, M0 = 10000, 2500
N1, M1 = 2500, 1000
NNZ0, NNZ1 = 320000, 40000
D = 128

NC, NS, LANES = 2, 16, 16
NW = NC * NS
CHUNK = 128

SP_N0, SP_M0, SP_N1, SP_M1 = 10240, 2560, 2560, 1024

K0 = 80
K1 = 16
KP = 8
P0 = NW * K0 * CHUNK
P1 = NW * K1 * CHUNK
PP = NW * KP * CHUNK


def _mesh():
    return plsc.VectorSubcoreMesh(core_axis_name="c", subcore_axis_name="s",
                                  num_cores=NC, num_subcores=NS)


@functools.lru_cache(None)
def _make_rowpass(k, sp):
    rpt = sp // NS
    ksub = min(k, 40)
    n_half = k // ksub

    @functools.partial(
        pl.kernel,
        out_type=jax.ShapeDtypeStruct((NC, sp, D), jnp.float32),
        mesh=_mesh(),
        scratch_types=[
            pltpu.VMEM((ksub, CHUNK), jnp.int32),
            pltpu.VMEM((ksub, CHUNK), jnp.int32),
            pltpu.VMEM((CHUNK, D), jnp.float32),
            pltpu.VMEM((CHUNK, D), jnp.float32),
            pltpu.VMEM_SHARED((sp, D), jnp.float32),
            pltpu.SemaphoreType.DMA,
            pltpu.SemaphoreType.DMA,
        ],
    )
    def rowpass(table, src2d, dst2d, zeros, out,
                idx_s, idx_d, rows0, rows1, acc, sem0, sem1):
        cid = lax.axis_index("c")
        sid = lax.axis_index("s")
        wid = sid * NC + cid
        pltpu.sync_copy(zeros.at[pl.ds(sid * rpt, rpt)],
                        acc.at[pl.ds(sid * rpt, rpt)])
        plsc.subcore_barrier()

        rows = (rows0, rows1)
        sems = (sem0, sem1)
        for h in range(n_half):
            pltpu.sync_copy(src2d.at[pl.ds(wid * k + h * ksub, ksub)], idx_s)
            pltpu.sync_copy(dst2d.at[pl.ds(wid * k + h * ksub, ksub)], idx_d)
            pltpu.async_copy(table.at[idx_s.at[0]], rows0, sem0)

            @pl.loop(0, ksub // 2)
            def _(i):
                for u in (0, 1):
                    t = i * 2 + u
                    pltpu.make_async_copy(table.at[idx_s.at[t]],
                                          rows[u], sems[u]).wait()

                    @pl.when(t + 1 < ksub)
                    def _():
                        pltpu.async_copy(table.at[idx_s.at[t + 1]],
                                         rows[1 - u], sems[1 - u])

                    pltpu.sync_copy(rows[u], acc.at[idx_d.at[t]], add=True)

        plsc.subcore_barrier()
        pltpu.sync_copy(acc.at[pl.ds(sid * rpt, rpt)],
                        out.at[cid, pl.ds(sid * rpt, rpt)])

    return rowpass


@functools.lru_cache(None)
def _make_rowpass4(k, sp):
    rpt = sp // NS

    @functools.partial(
        pl.kernel,
        out_type=jax.ShapeDtypeStruct((NC, sp, D), jnp.float32),
        mesh=_mesh(),
        scratch_types=[
            pltpu.VMEM((k, CHUNK), jnp.int32),
            pltpu.VMEM((k, CHUNK), jnp.int32),
            pltpu.VMEM((CHUNK, D), jnp.float32),
            pltpu.VMEM((CHUNK, D), jnp.float32),
            pltpu.VMEM((CHUNK, D), jnp.float32),
            pltpu.VMEM((CHUNK, D), jnp.float32),
            pltpu.VMEM_SHARED((sp, D), jnp.float32),
            pltpu.SemaphoreType.DMA,
            pltpu.SemaphoreType.DMA,
            pltpu.SemaphoreType.DMA,
            pltpu.SemaphoreType.DMA,
            pltpu.SemaphoreType.DMA,
            pltpu.SemaphoreType.DMA,
            pltpu.SemaphoreType.DMA,
            pltpu.SemaphoreType.DMA,
        ],
    )
    def rowpass(table, src2d, dst2d, zeros, out,
                idx_s, idx_d, r0, r1, r2, r3, acc,
                g0, g1, g2, g3, s0, s1, s2, s3):
        cid = lax.axis_index("c")
        sid = lax.axis_index("s")
        wid = sid * NC + cid
        rows = (r0, r1, r2, r3)
        gsem = (g0, g1, g2, g3)
        ssem = (s0, s1, s2, s3)
        pltpu.sync_copy(zeros.at[pl.ds(sid * rpt, rpt)],
                        acc.at[pl.ds(sid * rpt, rpt)])
        pltpu.sync_copy(src2d.at[pl.ds(wid * k, k)], idx_s)
        pltpu.sync_copy(dst2d.at[pl.ds(wid * k, k)], idx_d)
        plsc.subcore_barrier()

        pltpu.async_copy(table.at[idx_s.at[0]], rows[0], gsem[0])
        pltpu.async_copy(table.at[idx_s.at[1]], rows[1], gsem[1])

        @pl.loop(0, k // 4)
        def _(i):
            for u in range(4):
                t = i * 4 + u
                bf = (u + 2) % 4
                pltpu.make_async_copy(table.at[idx_s.at[t]],
                                      rows[u], gsem[u]).wait()
                pltpu.async_copy(rows[u], acc.at[idx_d.at[t]],
                                 ssem[u], add=True)

                @pl.when(t + 2 < k)
                def _():
                    @pl.when(t >= 2)
                    def _():
                        pltpu.make_async_copy(
                            rows[bf], acc.at[idx_d.at[t]], ssem[bf]).wait()

                    pltpu.async_copy(table.at[idx_s.at[t + 2]],
                                     rows[bf], gsem[bf])

        for b in range(4):
            pltpu.make_async_copy(rows[b], acc.at[idx_d.at[0]],
                                  ssem[b]).wait()
        plsc.subcore_barrier()
        pltpu.sync_copy(acc.at[pl.ds(sid * rpt, rpt)],
                        out.at[cid, pl.ds(sid * rpt, rpt)])

    return rowpass


_HISTS = ((SP_N0, P0), (SP_M0, P0), (SP_N1, P1), (SP_M1, P1), (SP_M0, PP))


@functools.lru_cache(None)
def _make_hist():
    pmax = max(p for _, p in _HISTS) // NS

    @functools.partial(
        pl.kernel,
        out_type=[jax.ShapeDtypeStruct((sp,), jnp.float32)
                  for sp, _ in _HISTS],
        mesh=_mesh(),
        compiler_params=pltpu.CompilerParams(needs_layout_passes=False),
        scratch_types=[
            pltpu.VMEM((pmax,), jnp.int32),
            pltpu.VMEM((max(sp for sp, _ in _HISTS) // NS,), jnp.float32),
            pltpu.VMEM((max(sp for sp, _ in _HISTS) // NS,), jnp.float32),
        ] + [pltpu.VMEM((sp,), jnp.float32) for sp, _ in _HISTS]
          + [pltpu.VMEM_SHARED((NS * sp,), jnp.float32) for sp, _ in _HISTS],
    )
    def hist(i0, i1, i2, i3, i4,
             o0, o1, o2, o3, o4,
             idxbuf, tmp, tmp2, a0, a1, a2, a3, a4,
             s0, s1, s2, s3, s4):
        cid = lax.axis_index("c")
        sid = lax.axis_index("s")
        ones_v = jnp.ones((LANES,), jnp.float32)
        zeros_v = jnp.zeros((LANES,), jnp.float32)
        accs = (a0, a1, a2, a3, a4)
        stages = (s0, s1, s2, s3, s4)
        outs = (o0, o1, o2, o3, o4)
        idxs = (i0, i1, i2, i3, i4)

        @pl.when(cid == 0)
        def _():
            for idx_flat, a, st, (sp, p) in zip(idxs, accs, stages, _HISTS):
                ppt = p // NS

                @pl.loop(0, sp // LANES)
                def _(v, a=a):
                    a[pl.ds(v * LANES, LANES)] = zeros_v

                pltpu.sync_copy(idx_flat.at[pl.ds(sid * ppt, ppt)],
                                idxbuf.at[pl.ds(0, ppt)])

                @pl.loop(0, ppt // LANES)
                def _(q, a=a):
                    vec = idxbuf[pl.ds(q * LANES, LANES)]
                    plsc.addupdate_scatter(a, [vec], ones_v)

                pltpu.sync_copy(a, st.at[pl.ds(sid * sp, sp)])

        plsc.subcore_barrier()

        @pl.when(cid == 0)
        def _():
            for st, o, (sp, _) in zip(stages, outs, _HISTS):
                spt = sp // NS
                pltpu.sync_copy(st.at[pl.ds(sid * spt, spt)],
                                tmp.at[pl.ds(0, spt)])
                for s in range(1, NS):
                    pltpu.sync_copy(st.at[pl.ds(s * sp + sid * spt, spt)],
                                    tmp2.at[pl.ds(0, spt)])

                    @pl.loop(0, spt // LANES)
                    def _(v):
                        sl = pl.ds(v * LANES, LANES)
                        tmp[sl] = tmp[sl] + tmp2[sl]

                pltpu.sync_copy(tmp.at[pl.ds(0, spt)],
                                o.at[pl.ds(sid * spt, spt)])

    return hist


@functools.lru_cache(None)
def _make_unpool(k):
    @functools.partial(
        pl.kernel,
        out_type=jax.ShapeDtypeStruct((NW * k * CHUNK, D), jnp.float32),
        mesh=_mesh(),
        scratch_types=[
            pltpu.VMEM((k, CHUNK), jnp.int32),
            pltpu.VMEM((CHUNK, D), jnp.float32),
            pltpu.SemaphoreType.DMA,
        ],
    )
    def unpool(table, src2d, out, idx_s, rows, sem):
        cid = lax.axis_index("c")
        sid = lax.axis_index("s")
        wid = sid * NC + cid
        pltpu.sync_copy(src2d.at[pl.ds(wid * k, k)], idx_s)

        @pl.loop(0, k)
        def _(j):
            pltpu.async_copy(table.at[idx_s.at[j]], rows, sem).wait()
            pltpu.sync_copy(rows, out.at[pl.ds((wid * k + j) * CHUNK, CHUNK)])

    return unpool


def _tc(body, out_shape, *args):
    return pl.pallas_call(body, out_shape=out_shape)(*args)


def _safe_rsqrt(c):
    return jnp.where(c > 0, lax.rsqrt(jnp.where(c > 0, c, 1.0)), 0.0)


def _safe_recip(c):
    return jnp.where(c > 0, 1.0 / jnp.where(c > 0, c, 1.0), 0.0)


def _comb_body(hn0, hh0, hn1, hh1, hasm, dvs0, dei0, dvs1, dei1, ci):
    dvs0[...] = _safe_rsqrt(hn0[...])
    dei0[...] = _safe_recip(hh0[...])
    dvs1[...] = _safe_rsqrt(hn1[...])
    dei1[...] = _safe_recip(hh1[...])
    ci[...] = _safe_recip(hasm[...])


def _mm_scale_body(x, w, b, s, o):
    o[...] = (jnp.dot(x[...], w[...], preferred_element_type=jnp.float32)
              + b[...]) * s[...]


def _comb_scale_body(p, s, o):
    o[...] = (p[0] + p[1]) * s[...]


def _comb_scale_relu_body(p, s, o):
    o[...] = jnp.maximum((p[0] + p[1]) * s[...], 0.0)


def _pool_mm_body(p, ci, w, b, s, o):
    xc = (p[0] + p[1]) * ci[...]
    o[...] = (jnp.dot(xc, w[...], preferred_element_type=jnp.float32)
              + b[...]) * s[...]


def _skip_mm_body(xu, h0, wa, wb, b, s, o):
    y = (jnp.dot(xu[0:SP_N0, :], wa[...], preferred_element_type=jnp.float32)
         + jnp.dot(h0[...], wb[...], preferred_element_type=jnp.float32)
         + b[...])
    o[...] = y * s[...]


def _final_body(p, s, w, b, o):
    h2 = jnp.maximum((p[0] + p[1]) * s[...], 0.0)
    o[...] = jnp.dot(h2, w[...], preferred_element_type=jnp.float32) + b[...]


def _f32(shape):
    return jax.ShapeDtypeStruct(shape, jnp.float32)


def _pad_idx(a, total, fill):
    pad = jnp.full((total - a.shape[0],), fill, jnp.int32)
    return jnp.concatenate([a.astype(jnp.int32), pad]).reshape(-1, CHUNK)


def kernel(X, W0, b0, W1, b1, W2, b2, W3, b3,
           inc_nodes0, inc_hedges0, inc_nodes1, inc_hedges1, asm_idx):
    f32 = jnp.float32
    zeros128 = jnp.zeros((SP_N0, D), f32)

    srcA0 = _pad_idx(inc_nodes0, P0, 0)
    dstA0 = _pad_idx(inc_hedges0, P0, M0)
    srcB0 = _pad_idx(inc_hedges0, P0, 0)
    dstB0 = _pad_idx(inc_nodes0, P0, N0)
    srcA1 = _pad_idx(inc_nodes1, P1, 0)
    dstA1 = _pad_idx(inc_hedges1, P1, M1)
    srcB1 = _pad_idx(inc_hedges1, P1, 0)
    dstB1 = _pad_idx(inc_nodes1, P1, N1)
    srcP = _pad_idx(jnp.arange(N0, dtype=jnp.int32), PP, 0)
    dstP = _pad_idx(asm_idx, PP, N1)
    srcU = _pad_idx(asm_idx, PP, 0)

    Xp = jnp.concatenate([X, jnp.zeros((SP_N0 - N0, D), f32)], axis=0)
    b0r, b1r, b2r, b3r = (b.reshape(1, D) for b in (b0, b1, b2, b3))

    hn0, hh0, hn1, hh1, hasm = _make_hist()(
        dstB0.reshape(-1), dstA0.reshape(-1), dstB1.reshape(-1),
        dstA1.reshape(-1), dstP.reshape(-1))
    dvs0, dei0, dvs1, dei1, ci = _tc(
        _comb_body,
        (_f32((SP_N0, 1)), _f32((SP_M0, 1)), _f32((SP_N1, 1)),
         _f32((SP_M1, 1)), _f32((SP_M0, 1))),
        hn0.reshape(-1, 1), hh0.reshape(-1, 1), hn1.reshape(-1, 1),
        hh1.reshape(-1, 1), hasm.reshape(-1, 1))

    zrows = lambda sp: zeros128[:sp]

    Xs0 = _tc(_mm_scale_body, _f32((SP_N0, D)), Xp, W0, b0r, dvs0)
    pA = _make_rowpass4(K0, SP_M0)(Xs0, srcA0, dstA0, zrows(SP_M0))
    he0 = _tc(_comb_scale_body, _f32((SP_M0, D)), pA, dei0)
    pB = _make_rowpass(K0, SP_N0)(he0, srcB0, dstB0, zrows(SP_N0))
    h0 = _tc(_comb_scale_relu_body, _f32((SP_N0, D)), pB, dvs0)

    pP = _make_rowpass4(KP, SP_M0)(h0, srcP, dstP, zrows(SP_M0))
    Xs1 = _tc(_pool_mm_body, _f32((SP_N1, D)), pP, ci, W1, b1r, dvs1)
    qA = _make_rowpass4(K1, SP_M1)(Xs1, srcA1, dstA1, zrows(SP_M1))
    he1 = _tc(_comb_scale_body, _f32((SP_M1, D)), qA, dei1)
    qB = _make_rowpass4(K1, SP_N1)(he1, srcB1, dstB1, zrows(SP_N1))
    hb = _tc(_comb_scale_relu_body, _f32((SP_N1, D)), qB, dvs1)

    Xu = _make_unpool(KP)(hb, srcU)
    W2a, W2b = W2[:D], W2[D:]
    Y2 = _tc(_skip_mm_body, _f32((SP_N0, D)), Xu, h0, W2a, W2b, b2r, dvs0)
    rA = _make_rowpass4(K0, SP_M0)(Y2, srcA0, dstA0, zrows(SP_M0))
    he2 = _tc(_comb_scale_body, _f32((SP_M0, D)), rA, dei0)
    rB = _make_rowpass(K0, SP_N0)(he2, srcB0, dstB0, zrows(SP_N0))
    out = _tc(_final_body, _f32((SP_N0, D)), rB, dvs0, W3, b3r)

    return out[:N0]
